# Initial kernel scaffold; baseline (speedup 1.0000x reference)
import jax
import jax.numpy as jnp
from jax.experimental import pallas as pl


def kernel(x, y, x_weights, y_weights):
    raise NotImplementedError("write your pallas kernel here")


# speedup vs baseline: 86.5784x; 86.5784x over previous
import jax
import jax.numpy as jnp
import numpy as np
from jax.experimental import pallas as pl
from jax.experimental.pallas import tpu as pltpu

LOG_R = 11
LOG_C = 7
R = 1 << LOG_R
C = 1 << LOG_C
N_LOG = LOG_R + LOG_C


def _pass_tables():
    js, ks = [], []
    for k in range(1, N_LOG + 1):
        for j in range(k - 1, -1, -1):
            js.append(j)
            ks.append(k)
    return np.asarray(js, np.int32), np.asarray(ks, np.int32)


_J_TAB, _K_TAB = _pass_tables()
N_PASS = len(_J_TAB)


def _wasserstein_body(jt_ref, kt_ref, x_ref, y_ref, xw_ref, yw_ref, out_ref, zb, sb):
    xw = xw_ref[...]
    yw = yw_ref[...]
    sx = jnp.sum(xw)
    sy = jnp.sum(yw)
    zb[0 : R // 2, :] = x_ref[...]
    zb[R // 2 : R, :] = y_ref[...]
    sb[0 : R // 2, :] = xw * (1.0 / sx)
    sb[R // 2 : R, :] = yw * (-1.0 / sy)

    riota = jax.lax.broadcasted_iota(jnp.int32, (R, C), 0)
    ciota = jax.lax.broadcasted_iota(jnp.int32, (R, C), 1)

    def cmpex(z, s, p, ps, b0, ascbit):
        want_min = (b0 ^ ascbit) == 0
        swap = (want_min & (p < z)) | (jnp.logical_not(want_min) & (p > z))
        return jnp.where(swap, p, z), jnp.where(swap, ps, s)

    def pass_body(i, _):
        j = jt_ref[i]
        k = kt_ref[i]
        z = zb[...]
        s = sb[...]
        ascbit = jnp.where(
            k < LOG_R,
            (riota >> k) & 1,
            (ciota >> jnp.maximum(k - LOG_R, 0)) & 1,
        )

        def sub_case(z, s):
            d = jnp.left_shift(1, j)
            b0 = 1 - ((riota >> j) & 1)
            zu = pltpu.roll(z, R - d, 0)
            zd = pltpu.roll(z, d, 0)
            su = pltpu.roll(s, R - d, 0)
            sd = pltpu.roll(s, d, 0)
            p = jnp.where(b0 == 1, zu, zd)
            ps = jnp.where(b0 == 1, su, sd)
            return cmpex(z, s, p, ps, 1 - b0, ascbit)

        def lane_case(z, s):
            jc = jnp.maximum(j - LOG_R, 0)
            d = jnp.left_shift(1, jc)
            b0 = 1 - ((ciota >> jc) & 1)
            zu = pltpu.roll(z, C - d, 1)
            zd = pltpu.roll(z, d, 1)
            su = pltpu.roll(s, C - d, 1)
            sd = pltpu.roll(s, d, 1)
            p = jnp.where(b0 == 1, zu, zd)
            ps = jnp.where(b0 == 1, su, sd)
            return cmpex(z, s, p, ps, 1 - b0, ascbit)

        zn, sn = jax.lax.cond(j >= LOG_R, lane_case, sub_case, z, s)
        zb[...] = zn
        sb[...] = sn
        return 0

    jax.lax.fori_loop(0, N_PASS, pass_body, 0)

    def cum_body(t, c):
        sh = jnp.left_shift(1, t)
        shifted = pltpu.roll(c, sh, 0)
        return c + jnp.where(riota >= sh, shifted, 0.0)

    c = jax.lax.fori_loop(0, LOG_R, cum_body, sb[...])
    tot = c[R - 1 : R, :]
    rj = jax.lax.broadcasted_iota(jnp.int32, (C, C), 0)
    cj = jax.lax.broadcasted_iota(jnp.int32, (C, C), 1)
    excl = jnp.sum(
        jnp.where(rj < cj, jnp.broadcast_to(tot.reshape(C, 1), (C, C)), 0.0),
        axis=0,
        keepdims=True,
    )
    c = c + excl

    z = zb[...]
    lane = jax.lax.broadcasted_iota(jnp.int32, (1, C), 1)
    row0_next = jnp.where(lane == C - 1, z[R - 1 : R, :], pltpu.roll(z[0:1, :], C - 1, 1))
    zn = jnp.concatenate([z[1:R], row0_next], axis=0)

    out_ref[0, 0] = jnp.sum(jnp.abs(c) * (zn - z))


@jax.jit
def kernel(x, y, x_weights, y_weights):
    x2 = x.reshape(R // 2, C)
    y2 = y.reshape(R // 2, C)
    xw2 = x_weights.reshape(R // 2, C)
    yw2 = y_weights.reshape(R // 2, C)
    out = pl.pallas_call(
        _wasserstein_body,
        out_shape=jax.ShapeDtypeStruct((1, 1), jnp.float32),
        in_specs=[
            pl.BlockSpec(memory_space=pltpu.SMEM),
            pl.BlockSpec(memory_space=pltpu.SMEM),
            pl.BlockSpec(memory_space=pltpu.VMEM),
            pl.BlockSpec(memory_space=pltpu.VMEM),
            pl.BlockSpec(memory_space=pltpu.VMEM),
            pl.BlockSpec(memory_space=pltpu.VMEM),
        ],
        out_specs=pl.BlockSpec(memory_space=pltpu.SMEM),
        scratch_shapes=[
            pltpu.VMEM((R, C), jnp.float32),
            pltpu.VMEM((R, C), jnp.float32),
        ],
    )(jnp.asarray(_J_TAB), jnp.asarray(_K_TAB), x2, y2, xw2, yw2)
    return out.reshape(())


# speedup vs baseline: 138.9942x; 1.6054x over previous
import jax
import jax.numpy as jnp
from jax import lax
from jax.experimental import pallas as pl
from jax.experimental.pallas import tpu as pltpu
from jax.experimental.pallas import tpu_sc as plsc

N = 131072
N2 = 2 * N
NW = 16
CHUNK = N2 // NW
VREGS = CHUNK // 16
NPASS = 7
RADIX = 32
PAD = 128
MINI = -2147483648


def _srl(v, sh):
    return lax.shift_right_logical(v, sh)


def _body(x_hbm, y_hbm, xw_hbm, yw_hbm, out_hbm, hk_hbm, hv_hbm,
          key_v, val_v, oidx_v, kstage_v, vstage_v, hist_v, base_v, tmp_v, itmp_v, scal_v,
          dstk_s, dstv_s, grid_s, part_s):
    wid = lax.axis_index("s")
    base = wid * CHUNK
    lane = lax.iota(jnp.int32, 16)

    half = wid < (NW // 2)
    src_off = jnp.where(half, base, base - N)

    @pl.when(half)
    def _():
        pltpu.sync_copy(x_hbm.at[pl.ds(src_off, CHUNK)], vstage_v)
        pltpu.sync_copy(xw_hbm.at[pl.ds(src_off, CHUNK)], val_v)

    @pl.when(jnp.logical_not(half))
    def _():
        pltpu.sync_copy(y_hbm.at[pl.ds(src_off, CHUNK)], vstage_v)
        pltpu.sync_copy(yw_hbm.at[pl.ds(src_off, CHUNK)], val_v)

    sign = jnp.where(half, 1.0, -1.0)

    @pl.loop(0, VREGS)
    def _(i):
        zb = plsc.bitcast(vstage_v[pl.ds(i * 16, 16)], jnp.int32)
        mono = jnp.where(zb < 0, ~zb, zb ^ jnp.int32(MINI))
        key_v[pl.ds(i * 16, 16)] = mono
        val_v[pl.ds(i * 16, 16)] = val_v[pl.ds(i * 16, 16)] * sign

    wsum = lax.fori_loop(
        0, VREGS, lambda i, a: a + val_v[pl.ds(i * 16, 16)],
        jnp.zeros((16,), jnp.float32))
    tmp_v[pl.ds(0, 16)] = jnp.zeros((16,), jnp.float32) + jnp.sum(wsum) * sign
    pltpu.sync_copy(tmp_v.at[pl.ds(0, 16)], part_s.at[pl.ds(wid * 16, 16)])

    pltpu.sync_copy(key_v.at[pl.ds(0, CHUNK)], hk_hbm.at[pl.ds(base, CHUNK)])
    pltpu.sync_copy(val_v, hv_hbm.at[pl.ds(base, CHUNK)])
    plsc.subcore_barrier()

    def radix_pass(p):
        sh = 5 * p
        pltpu.sync_copy(hk_hbm.at[pl.ds(base, CHUNK)], key_v.at[pl.ds(0, CHUNK)])
        pltpu.sync_copy(hv_hbm.at[pl.ds(base, CHUNK)], val_v)

        @pl.loop(0, RADIX)
        def _(i):
            hist_v[pl.ds(i * 16, 16)] = jnp.zeros((16,), jnp.int32)

        ones = jnp.ones((16,), jnp.int32)

        @pl.loop(0, VREGS)
        def _(i):
            k = key_v[pl.ds(i * 16, 16)]
            d = _srl(k, sh) & 31
            plsc.addupdate_scatter(hist_v, [lane * 32 + d], ones)

        clo = jnp.zeros((16,), jnp.int32)
        chi = jnp.zeros((16,), jnp.int32)
        for l in range(16):
            clo = clo + hist_v[pl.ds(l * 32, 16)]
            chi = chi + hist_v[pl.ds(l * 32 + 16, 16)]
        itmp_v[pl.ds(0, 16)] = clo
        itmp_v[pl.ds(16, 16)] = chi
        pltpu.sync_copy(itmp_v.at[pl.ds(0, 32)], grid_s.at[pl.ds(wid * 32, 32)])
        plsc.subcore_barrier()

        pltpu.sync_copy(grid_s, itmp_v)
        tot_lo = jnp.zeros((16,), jnp.int32)
        tot_hi = jnp.zeros((16,), jnp.int32)
        bef_lo = jnp.zeros((16,), jnp.int32)
        bef_hi = jnp.zeros((16,), jnp.int32)
        for w in range(NW):
            g_lo = itmp_v[pl.ds(w * 32, 16)]
            g_hi = itmp_v[pl.ds(w * 32 + 16, 16)]
            tot_lo = tot_lo + g_lo
            tot_hi = tot_hi + g_hi
            m = w < wid
            bef_lo = bef_lo + jnp.where(m, g_lo, 0)
            bef_hi = bef_hi + jnp.where(m, g_hi, 0)
        ex_lo = plsc.cumsum(tot_lo) - tot_lo
        ex_hi = plsc.cumsum(tot_hi) - tot_hi + jnp.sum(tot_lo)
        base_v[pl.ds(0, 16)] = ex_lo + bef_lo
        base_v[pl.ds(16, 16)] = ex_hi + bef_hi

        @pl.loop(0, VREGS)
        def _(i):
            kv = key_v[pl.ds(i * 16, 16)]
            vv = val_v[pl.ds(i * 16, 16)]
            d = _srl(kv, sh) & 31
            dk, lv = plsc.sort_key_val(d * 16 + lane, lane)
            ds_ = _srl(dk, 4)
            prev = ds_.at[jnp.maximum(lane - 1, 0)].get(mode="promise_in_bounds")
            nxt = ds_.at[jnp.minimum(lane + 1, 15)].get(mode="promise_in_bounds")
            is_new = (ds_ != prev) | (lane == 0)
            is_last = (ds_ != nxt) | (lane == 15)
            segstart = plsc.cummax(jnp.where(is_new, lane, 0))
            rank = lane - segstart
            pos = plsc.load_gather(base_v, [ds_]) + rank
            plsc.addupdate_scatter(base_v, [ds_], rank + 1, mask=is_last)
            row = _srl(i, 3)
            col = (i & 7) * 16
            kstage_v[pl.ds(i * 16, 16)] = kv.at[lv].get(mode="promise_in_bounds")
            vstage_v[pl.ds(i * 16, 16)] = vv.at[lv].get(mode="promise_in_bounds")
            oidx_v[row, pl.ds(col, 16)] = pos

        @pl.loop(0, CHUNK // 128)
        def _(j):
            pltpu.sync_copy(kstage_v.at[pl.ds(j * 128, 128)], dstk_s.at[oidx_v.at[j]])
            pltpu.sync_copy(vstage_v.at[pl.ds(j * 128, 128)], dstv_s.at[oidx_v.at[j]])

        plsc.subcore_barrier()
        if p < NPASS - 1:
            pltpu.sync_copy(dstk_s.at[pl.ds(base, CHUNK)], hk_hbm.at[pl.ds(base, CHUNK)])
            pltpu.sync_copy(dstv_s.at[pl.ds(base, CHUNK)], hv_hbm.at[pl.ds(base, CHUNK)])

    for p in range(NPASS):
        radix_pass(p)

    pltpu.sync_copy(part_s, tmp_v.at[pl.ds(0, 256)])
    sx = jnp.zeros((16,), jnp.float32)
    sy = jnp.zeros((16,), jnp.float32)
    for w in range(NW):
        pv = tmp_v[pl.ds(w * 16, 16)]
        if w < NW // 2:
            sx = sx + pv
        else:
            sy = sy - pv
    rsx = 16.0 / (jnp.zeros((16,), jnp.float32) + jnp.sum(sx))
    rsy = -16.0 / (jnp.zeros((16,), jnp.float32) + jnp.sum(sy))
    plsc.subcore_barrier()

    pltpu.sync_copy(dstk_s.at[pl.ds(base, CHUNK + 16)], key_v)
    pltpu.sync_copy(dstv_s.at[pl.ds(base, CHUNK)], val_v)

    def norm_body(i, a):
        v = val_v[pl.ds(i * 16, 16)]
        v = jnp.where(v >= 0.0, v * rsx, v * rsy)
        val_v[pl.ds(i * 16, 16)] = v
        return a + v

    tloc = lax.fori_loop(0, VREGS, norm_body, jnp.zeros((16,), jnp.float32))
    tmp_v[pl.ds(0, 16)] = jnp.zeros((16,), jnp.float32) + jnp.sum(tloc)
    pltpu.sync_copy(tmp_v.at[pl.ds(0, 16)], part_s.at[pl.ds(wid * 16, 16)])
    plsc.subcore_barrier()

    pltpu.sync_copy(part_s, tmp_v.at[pl.ds(0, 256)])
    off = jnp.zeros((16,), jnp.float32)
    for w in range(NW):
        off = off + jnp.where(w < wid, tmp_v[pl.ds(w * 16, 16)], 0.0)
    off_sc = (jnp.zeros((16,), jnp.float32) + jnp.sum(off)) * 0.0625

    def unkey(vk):
        m = jnp.where(vk < 0, jnp.int32(MINI), jnp.int32(-1))
        return plsc.bitcast(vk ^ m, jnp.float32)

    def red_body(i, carry):
        run, acc = carry
        v = val_v[pl.ds(i * 16, 16)]
        cs = plsc.cumsum(v) + run
        run2 = jnp.zeros((16,), jnp.float32) + cs[15]
        z = unkey(key_v[pl.ds(i * 16, 16)])
        zn = unkey(plsc.load_gather(key_v, [lane + (i * 16 + 1)]))
        gi = (i * 16 + base) + lane
        dz = jnp.where(gi < N2 - 1, zn - z, 0.0)
        return (run2, acc + jnp.abs(cs + off_sc) * dz)

    _, accv = lax.fori_loop(
        0, VREGS, red_body,
        (jnp.zeros((16,), jnp.float32), jnp.zeros((16,), jnp.float32)))
    tmp_v[pl.ds(0, 16)] = jnp.zeros((16,), jnp.float32) + jnp.sum(accv)
    pltpu.sync_copy(tmp_v.at[pl.ds(0, 16)], part_s.at[pl.ds(wid * 16, 16)])
    plsc.subcore_barrier()

    @pl.when(wid == 0)
    def _():
        pltpu.sync_copy(part_s, tmp_v.at[pl.ds(0, 256)])
        t = jnp.zeros((16,), jnp.float32)
        for w in range(NW):
            t = t + tmp_v[pl.ds(w * 16, 16)]
        scal_v[pl.ds(0, 16)] = (jnp.zeros((16,), jnp.float32) + jnp.sum(t)) * 0.0625
        pltpu.sync_copy(scal_v.at[pl.ds(0, 16)], out_hbm)


@jax.jit
def kernel(x, y, x_weights, y_weights):
    mesh = plsc.VectorSubcoreMesh(core_axis_name="c", subcore_axis_name="s",
                                  num_cores=1)
    run = pl.kernel(
        _body,
        out_type=(jax.ShapeDtypeStruct((16,), jnp.float32),
                  jax.ShapeDtypeStruct((N2,), jnp.int32),
                  jax.ShapeDtypeStruct((N2,), jnp.float32)),
        mesh=mesh,
        compiler_params=pltpu.CompilerParams(needs_layout_passes=False),
        scratch_types=[
            pltpu.VMEM((CHUNK + 16,), jnp.int32),
            pltpu.VMEM((CHUNK,), jnp.float32),
            pltpu.VMEM((CHUNK // 128, 128), jnp.int32),
            pltpu.VMEM((CHUNK,), jnp.int32),
            pltpu.VMEM((CHUNK,), jnp.float32),
            pltpu.VMEM((RADIX * 16,), jnp.int32),
            pltpu.VMEM((RADIX,), jnp.int32),
            pltpu.VMEM((512,), jnp.float32),
            pltpu.VMEM((512,), jnp.int32),
            pltpu.VMEM((16,), jnp.float32),
            pltpu.VMEM_SHARED((N2 + PAD,), jnp.int32),
            pltpu.VMEM_SHARED((N2,), jnp.float32),
            pltpu.VMEM_SHARED((NW * 32,), jnp.int32),
            pltpu.VMEM_SHARED((NW * 16,), jnp.float32),
        ],
    )
    out, _, _ = run(x, y, x_weights, y_weights)
    return out[0].reshape(())


# speedup vs baseline: 145.8325x; 1.0492x over previous
import jax
import jax.numpy as jnp
from jax import lax
from jax.experimental import pallas as pl
from jax.experimental.pallas import tpu as pltpu
from jax.experimental.pallas import tpu_sc as plsc

N = 131072
N2 = 2 * N
NW = 16
CHUNK = N2 // NW
VREGS = CHUNK // 16
NPASS = 7
RADIX = 32
PAD = 128
MINI = -2147483648


def _srl(v, sh):
    return lax.shift_right_logical(v, sh)


def _body(x_hbm, y_hbm, xw_hbm, yw_hbm, out_hbm,
          key_v, val_v, oidx_v, kstage_v, vstage_v, hist_v, base_v, tmp_v, itmp_v, scal_v,
          dstk_s, dstv_s, grid_s, part_s):
    wid = lax.axis_index("s")
    base = wid * CHUNK
    lane = lax.iota(jnp.int32, 16)

    half = wid < (NW // 2)
    src_off = jnp.where(half, base, base - N)

    @pl.when(half)
    def _():
        pltpu.sync_copy(x_hbm.at[pl.ds(src_off, CHUNK)], vstage_v)
        pltpu.sync_copy(xw_hbm.at[pl.ds(src_off, CHUNK)], val_v)

    @pl.when(jnp.logical_not(half))
    def _():
        pltpu.sync_copy(y_hbm.at[pl.ds(src_off, CHUNK)], vstage_v)
        pltpu.sync_copy(yw_hbm.at[pl.ds(src_off, CHUNK)], val_v)

    sign = jnp.where(half, 1.0, -1.0)

    @pl.loop(0, VREGS)
    def _(i):
        zb = plsc.bitcast(vstage_v[pl.ds(i * 16, 16)], jnp.int32)
        mono = jnp.where(zb < 0, ~zb, zb ^ jnp.int32(MINI))
        key_v[pl.ds(i * 16, 16)] = mono
        val_v[pl.ds(i * 16, 16)] = val_v[pl.ds(i * 16, 16)] * sign

    wsum = lax.fori_loop(
        0, VREGS, lambda i, a: a + val_v[pl.ds(i * 16, 16)],
        jnp.zeros((16,), jnp.float32))
    tmp_v[pl.ds(0, 16)] = jnp.zeros((16,), jnp.float32) + jnp.sum(wsum) * sign
    pltpu.sync_copy(tmp_v.at[pl.ds(0, 16)], part_s.at[pl.ds(wid * 16, 16)])

    pltpu.sync_copy(key_v.at[pl.ds(0, CHUNK)], dstk_s.at[pl.ds(base, CHUNK)])
    pltpu.sync_copy(val_v, dstv_s.at[pl.ds(base, CHUNK)])
    plsc.subcore_barrier()

    def radix_pass(p):
        sh = 5 * p
        pltpu.sync_copy(dstk_s.at[pl.ds(base, CHUNK)], key_v.at[pl.ds(0, CHUNK)])
        pltpu.sync_copy(dstv_s.at[pl.ds(base, CHUNK)], val_v)

        @pl.loop(0, RADIX)
        def _(i):
            hist_v[pl.ds(i * 16, 16)] = jnp.zeros((16,), jnp.int32)

        ones = jnp.ones((16,), jnp.int32)

        @pl.loop(0, VREGS)
        def _(i):
            k = key_v[pl.ds(i * 16, 16)]
            d = _srl(k, sh) & 31
            plsc.addupdate_scatter(hist_v, [lane * 32 + d], ones)

        clo = jnp.zeros((16,), jnp.int32)
        chi = jnp.zeros((16,), jnp.int32)
        for l in range(16):
            clo = clo + hist_v[pl.ds(l * 32, 16)]
            chi = chi + hist_v[pl.ds(l * 32 + 16, 16)]
        itmp_v[pl.ds(0, 16)] = clo
        itmp_v[pl.ds(16, 16)] = chi
        pltpu.sync_copy(itmp_v.at[pl.ds(0, 32)], grid_s.at[pl.ds(wid * 32, 32)])
        plsc.subcore_barrier()

        pltpu.sync_copy(grid_s, itmp_v)
        tot_lo = jnp.zeros((16,), jnp.int32)
        tot_hi = jnp.zeros((16,), jnp.int32)
        bef_lo = jnp.zeros((16,), jnp.int32)
        bef_hi = jnp.zeros((16,), jnp.int32)
        for w in range(NW):
            g_lo = itmp_v[pl.ds(w * 32, 16)]
            g_hi = itmp_v[pl.ds(w * 32 + 16, 16)]
            tot_lo = tot_lo + g_lo
            tot_hi = tot_hi + g_hi
            m = w < wid
            bef_lo = bef_lo + jnp.where(m, g_lo, 0)
            bef_hi = bef_hi + jnp.where(m, g_hi, 0)
        ex_lo = plsc.cumsum(tot_lo) - tot_lo
        ex_hi = plsc.cumsum(tot_hi) - tot_hi + jnp.sum(tot_lo)
        base_v[pl.ds(0, 16)] = ex_lo + bef_lo
        base_v[pl.ds(16, 16)] = ex_hi + bef_hi

        @pl.loop(0, VREGS)
        def _(i):
            kv = key_v[pl.ds(i * 16, 16)]
            vv = val_v[pl.ds(i * 16, 16)]
            d = _srl(kv, sh) & 31
            dk, lv = plsc.sort_key_val(d * 16 + lane, lane)
            ds_ = _srl(dk, 4)
            prev = ds_.at[jnp.maximum(lane - 1, 0)].get(mode="promise_in_bounds")
            nxt = ds_.at[jnp.minimum(lane + 1, 15)].get(mode="promise_in_bounds")
            is_new = (ds_ != prev) | (lane == 0)
            is_last = (ds_ != nxt) | (lane == 15)
            segstart = plsc.cummax(jnp.where(is_new, lane, 0))
            rank = lane - segstart
            pos = plsc.load_gather(base_v, [ds_]) + rank
            plsc.addupdate_scatter(base_v, [ds_], rank + 1, mask=is_last)
            row = _srl(i, 3)
            col = (i & 7) * 16
            kstage_v[pl.ds(i * 16, 16)] = kv.at[lv].get(mode="promise_in_bounds")
            vstage_v[pl.ds(i * 16, 16)] = vv.at[lv].get(mode="promise_in_bounds")
            oidx_v[row, pl.ds(col, 16)] = pos

        @pl.loop(0, CHUNK // 128)
        def _(j):
            pltpu.sync_copy(kstage_v.at[pl.ds(j * 128, 128)], dstk_s.at[oidx_v.at[j]])
            pltpu.sync_copy(vstage_v.at[pl.ds(j * 128, 128)], dstv_s.at[oidx_v.at[j]])

        plsc.subcore_barrier()

    for p in range(NPASS):
        radix_pass(p)

    pltpu.sync_copy(part_s, tmp_v.at[pl.ds(0, 256)])
    sx = jnp.zeros((16,), jnp.float32)
    sy = jnp.zeros((16,), jnp.float32)
    for w in range(NW):
        pv = tmp_v[pl.ds(w * 16, 16)]
        if w < NW // 2:
            sx = sx + pv
        else:
            sy = sy - pv
    rsx = 16.0 / (jnp.zeros((16,), jnp.float32) + jnp.sum(sx))
    rsy = -16.0 / (jnp.zeros((16,), jnp.float32) + jnp.sum(sy))
    plsc.subcore_barrier()

    pltpu.sync_copy(dstk_s.at[pl.ds(base, CHUNK + 16)], key_v)
    pltpu.sync_copy(dstv_s.at[pl.ds(base, CHUNK)], val_v)

    def norm_body(i, a):
        v = val_v[pl.ds(i * 16, 16)]
        v = jnp.where(v >= 0.0, v * rsx, v * rsy)
        val_v[pl.ds(i * 16, 16)] = v
        return a + v

    tloc = lax.fori_loop(0, VREGS, norm_body, jnp.zeros((16,), jnp.float32))
    tmp_v[pl.ds(0, 16)] = jnp.zeros((16,), jnp.float32) + jnp.sum(tloc)
    pltpu.sync_copy(tmp_v.at[pl.ds(0, 16)], part_s.at[pl.ds(wid * 16, 16)])
    plsc.subcore_barrier()

    pltpu.sync_copy(part_s, tmp_v.at[pl.ds(0, 256)])
    off = jnp.zeros((16,), jnp.float32)
    for w in range(NW):
        off = off + jnp.where(w < wid, tmp_v[pl.ds(w * 16, 16)], 0.0)
    off_sc = (jnp.zeros((16,), jnp.float32) + jnp.sum(off)) * 0.0625

    def unkey(vk):
        m = jnp.where(vk < 0, jnp.int32(MINI), jnp.int32(-1))
        return plsc.bitcast(vk ^ m, jnp.float32)

    def red_body(i, carry):
        run, acc = carry
        v = val_v[pl.ds(i * 16, 16)]
        cs = plsc.cumsum(v) + run
        run2 = jnp.zeros((16,), jnp.float32) + cs[15]
        z = unkey(key_v[pl.ds(i * 16, 16)])
        zn = unkey(plsc.load_gather(key_v, [lane + (i * 16 + 1)]))
        gi = (i * 16 + base) + lane
        dz = jnp.where(gi < N2 - 1, zn - z, 0.0)
        return (run2, acc + jnp.abs(cs + off_sc) * dz)

    _, accv = lax.fori_loop(
        0, VREGS, red_body,
        (jnp.zeros((16,), jnp.float32), jnp.zeros((16,), jnp.float32)))
    tmp_v[pl.ds(0, 16)] = jnp.zeros((16,), jnp.float32) + jnp.sum(accv)
    pltpu.sync_copy(tmp_v.at[pl.ds(0, 16)], part_s.at[pl.ds(wid * 16, 16)])
    plsc.subcore_barrier()

    @pl.when(wid == 0)
    def _():
        pltpu.sync_copy(part_s, tmp_v.at[pl.ds(0, 256)])
        t = jnp.zeros((16,), jnp.float32)
        for w in range(NW):
            t = t + tmp_v[pl.ds(w * 16, 16)]
        scal_v[pl.ds(0, 16)] = (jnp.zeros((16,), jnp.float32) + jnp.sum(t)) * 0.0625
        pltpu.sync_copy(scal_v.at[pl.ds(0, 16)], out_hbm)


@jax.jit
def kernel(x, y, x_weights, y_weights):
    mesh = plsc.VectorSubcoreMesh(core_axis_name="c", subcore_axis_name="s",
                                  num_cores=1)
    run = pl.kernel(
        _body,
        out_type=jax.ShapeDtypeStruct((16,), jnp.float32),
        mesh=mesh,
        compiler_params=pltpu.CompilerParams(needs_layout_passes=False),
        scratch_types=[
            pltpu.VMEM((CHUNK + 16,), jnp.int32),
            pltpu.VMEM((CHUNK,), jnp.float32),
            pltpu.VMEM((CHUNK // 128, 128), jnp.int32),
            pltpu.VMEM((CHUNK,), jnp.int32),
            pltpu.VMEM((CHUNK,), jnp.float32),
            pltpu.VMEM((RADIX * 16,), jnp.int32),
            pltpu.VMEM((RADIX,), jnp.int32),
            pltpu.VMEM((512,), jnp.float32),
            pltpu.VMEM((512,), jnp.int32),
            pltpu.VMEM((16,), jnp.float32),
            pltpu.VMEM_SHARED((N2 + PAD,), jnp.int32),
            pltpu.VMEM_SHARED((N2,), jnp.float32),
            pltpu.VMEM_SHARED((NW * 32,), jnp.int32),
            pltpu.VMEM_SHARED((NW * 16,), jnp.float32),
        ],
    )
    out = run(x, y, x_weights, y_weights)
    return out[0].reshape(())


# speedup vs baseline: 183.1772x; 1.2561x over previous
import jax
import jax.numpy as jnp
from jax import lax
from jax.experimental import pallas as pl
from jax.experimental.pallas import tpu as pltpu
from jax.experimental.pallas import tpu_sc as plsc

N = 131072
N2 = 2 * N
NW = 16
CHUNK = N2 // NW
VREGS = CHUNK // 16
NPASS = 7
RADIX = 32
PAD = 128
MINI = -2147483648


def _srl(v, sh):
    return lax.shift_right_logical(v, sh)


def _body(x_hbm, y_hbm, xw_hbm, yw_hbm, out_hbm,
          key_v, val_v, oidx_v, kstage_v, vstage_v, hist_v, base_v, tmp_v, itmp_v, scal_v,
          dstk_s, dstv_s, grid_s, part_s, dma_sem):
    wid = lax.axis_index("s")
    base = wid * CHUNK
    lane = lax.iota(jnp.int32, 16)

    half = wid < (NW // 2)
    src_off = jnp.where(half, base, base - N)

    @pl.when(half)
    def _():
        pltpu.sync_copy(x_hbm.at[pl.ds(src_off, CHUNK)], vstage_v)
        pltpu.sync_copy(xw_hbm.at[pl.ds(src_off, CHUNK)], val_v)

    @pl.when(jnp.logical_not(half))
    def _():
        pltpu.sync_copy(y_hbm.at[pl.ds(src_off, CHUNK)], vstage_v)
        pltpu.sync_copy(yw_hbm.at[pl.ds(src_off, CHUNK)], val_v)

    sign = jnp.where(half, 1.0, -1.0)

    @pl.loop(0, VREGS, unroll=4)
    def _(i):
        zb = plsc.bitcast(vstage_v[pl.ds(i * 16, 16)], jnp.int32)
        mono = jnp.where(zb < 0, ~zb, zb ^ jnp.int32(MINI))
        key_v[pl.ds(i * 16, 16)] = mono
        val_v[pl.ds(i * 16, 16)] = val_v[pl.ds(i * 16, 16)] * sign

    wsum = lax.fori_loop(
        0, VREGS, lambda i, a: a + val_v[pl.ds(i * 16, 16)],
        jnp.zeros((16,), jnp.float32))
    tmp_v[pl.ds(0, 16)] = jnp.zeros((16,), jnp.float32) + jnp.sum(wsum) * sign
    pltpu.sync_copy(tmp_v.at[pl.ds(0, 16)], part_s.at[pl.ds(wid * 16, 16)])

    pltpu.sync_copy(key_v.at[pl.ds(0, CHUNK)], dstk_s.at[pl.ds(base, CHUNK)])
    pltpu.sync_copy(val_v, dstv_s.at[pl.ds(base, CHUNK)])
    plsc.subcore_barrier()

    def radix_pass(p):
        sh = 5 * p
        pltpu.sync_copy(dstk_s.at[pl.ds(base, CHUNK)], key_v.at[pl.ds(0, CHUNK)])
        pltpu.sync_copy(dstv_s.at[pl.ds(base, CHUNK)], val_v)

        @pl.loop(0, RADIX)
        def _(i):
            hist_v[pl.ds(i * 16, 16)] = jnp.zeros((16,), jnp.int32)

        ones = jnp.ones((16,), jnp.int32)

        @pl.loop(0, VREGS, unroll=4)
        def _(i):
            k = key_v[pl.ds(i * 16, 16)]
            d = _srl(k, sh) & 31
            plsc.addupdate_scatter(hist_v, [lane * 32 + d], ones)

        clo = jnp.zeros((16,), jnp.int32)
        chi = jnp.zeros((16,), jnp.int32)
        for l in range(16):
            clo = clo + hist_v[pl.ds(l * 32, 16)]
            chi = chi + hist_v[pl.ds(l * 32 + 16, 16)]
        itmp_v[pl.ds(0, 16)] = clo
        itmp_v[pl.ds(16, 16)] = chi
        pltpu.sync_copy(itmp_v.at[pl.ds(0, 32)], grid_s.at[pl.ds(wid * 32, 32)])
        plsc.subcore_barrier()

        pltpu.sync_copy(grid_s, itmp_v)
        tot_lo = jnp.zeros((16,), jnp.int32)
        tot_hi = jnp.zeros((16,), jnp.int32)
        bef_lo = jnp.zeros((16,), jnp.int32)
        bef_hi = jnp.zeros((16,), jnp.int32)
        for w in range(NW):
            g_lo = itmp_v[pl.ds(w * 32, 16)]
            g_hi = itmp_v[pl.ds(w * 32 + 16, 16)]
            tot_lo = tot_lo + g_lo
            tot_hi = tot_hi + g_hi
            m = w < wid
            bef_lo = bef_lo + jnp.where(m, g_lo, 0)
            bef_hi = bef_hi + jnp.where(m, g_hi, 0)
        ex_lo = plsc.cumsum(tot_lo) - tot_lo
        ex_hi = plsc.cumsum(tot_hi) - tot_hi + jnp.sum(tot_lo)
        base_v[pl.ds(0, 16)] = ex_lo + bef_lo
        base_v[pl.ds(16, 16)] = ex_hi + bef_hi

        @pl.loop(0, VREGS, unroll=4)
        def _(i):
            kv = key_v[pl.ds(i * 16, 16)]
            vv = val_v[pl.ds(i * 16, 16)]
            d = _srl(kv, sh) & 31
            dk, lv = plsc.sort_key_val(d * 16 + lane, lane)
            ds_ = _srl(dk, 4)
            prev = ds_.at[jnp.maximum(lane - 1, 0)].get(mode="promise_in_bounds")
            nxt = ds_.at[jnp.minimum(lane + 1, 15)].get(mode="promise_in_bounds")
            is_new = (ds_ != prev) | (lane == 0)
            is_last = (ds_ != nxt) | (lane == 15)
            segstart = plsc.cummax(jnp.where(is_new, lane, 0))
            rank = lane - segstart
            pos = plsc.load_gather(base_v, [ds_]) + rank
            plsc.addupdate_scatter(base_v, [ds_], rank + 1, mask=is_last)
            row = _srl(i, 3)
            col = (i & 7) * 16
            kstage_v[pl.ds(i * 16, 16)] = kv.at[lv].get(mode="promise_in_bounds")
            vstage_v[pl.ds(i * 16, 16)] = vv.at[lv].get(mode="promise_in_bounds")
            oidx_v[row, pl.ds(col, 16)] = pos

        @pl.loop(0, CHUNK // 128, step=8)
        def _(j):
            handles = []
            for u in range(8):
                handles.append(pltpu.async_copy(
                    kstage_v.at[pl.ds((j + u) * 128, 128)],
                    dstk_s.at[oidx_v.at[j + u]], dma_sem))
                handles.append(pltpu.async_copy(
                    vstage_v.at[pl.ds((j + u) * 128, 128)],
                    dstv_s.at[oidx_v.at[j + u]], dma_sem))
            for h in handles:
                h.wait()

        plsc.subcore_barrier()

    for p in range(NPASS):
        radix_pass(p)

    pltpu.sync_copy(part_s, tmp_v.at[pl.ds(0, 256)])
    sx = jnp.zeros((16,), jnp.float32)
    sy = jnp.zeros((16,), jnp.float32)
    for w in range(NW):
        pv = tmp_v[pl.ds(w * 16, 16)]
        if w < NW // 2:
            sx = sx + pv
        else:
            sy = sy - pv
    rsx = 16.0 / (jnp.zeros((16,), jnp.float32) + jnp.sum(sx))
    rsy = -16.0 / (jnp.zeros((16,), jnp.float32) + jnp.sum(sy))
    plsc.subcore_barrier()

    pltpu.sync_copy(dstk_s.at[pl.ds(base, CHUNK + 16)], key_v)
    pltpu.sync_copy(dstv_s.at[pl.ds(base, CHUNK)], val_v)

    def norm_body(i, a):
        v = val_v[pl.ds(i * 16, 16)]
        v = jnp.where(v >= 0.0, v * rsx, v * rsy)
        val_v[pl.ds(i * 16, 16)] = v
        return a + v

    tloc = lax.fori_loop(0, VREGS, norm_body, jnp.zeros((16,), jnp.float32))
    tmp_v[pl.ds(0, 16)] = jnp.zeros((16,), jnp.float32) + jnp.sum(tloc)
    pltpu.sync_copy(tmp_v.at[pl.ds(0, 16)], part_s.at[pl.ds(wid * 16, 16)])
    plsc.subcore_barrier()

    pltpu.sync_copy(part_s, tmp_v.at[pl.ds(0, 256)])
    off = jnp.zeros((16,), jnp.float32)
    for w in range(NW):
        off = off + jnp.where(w < wid, tmp_v[pl.ds(w * 16, 16)], 0.0)
    off_sc = (jnp.zeros((16,), jnp.float32) + jnp.sum(off)) * 0.0625

    def unkey(vk):
        m = jnp.where(vk < 0, jnp.int32(MINI), jnp.int32(-1))
        return plsc.bitcast(vk ^ m, jnp.float32)

    def red_body(i, carry):
        run, acc = carry
        v = val_v[pl.ds(i * 16, 16)]
        cs = plsc.cumsum(v) + run
        run2 = jnp.zeros((16,), jnp.float32) + cs[15]
        z = unkey(key_v[pl.ds(i * 16, 16)])
        zn = unkey(plsc.load_gather(key_v, [lane + (i * 16 + 1)]))
        gi = (i * 16 + base) + lane
        dz = jnp.where(gi < N2 - 1, zn - z, 0.0)
        return (run2, acc + jnp.abs(cs + off_sc) * dz)

    _, accv = lax.fori_loop(
        0, VREGS, red_body,
        (jnp.zeros((16,), jnp.float32), jnp.zeros((16,), jnp.float32)))
    tmp_v[pl.ds(0, 16)] = jnp.zeros((16,), jnp.float32) + jnp.sum(accv)
    pltpu.sync_copy(tmp_v.at[pl.ds(0, 16)], part_s.at[pl.ds(wid * 16, 16)])
    plsc.subcore_barrier()

    @pl.when(wid == 0)
    def _():
        pltpu.sync_copy(part_s, tmp_v.at[pl.ds(0, 256)])
        t = jnp.zeros((16,), jnp.float32)
        for w in range(NW):
            t = t + tmp_v[pl.ds(w * 16, 16)]
        scal_v[pl.ds(0, 16)] = (jnp.zeros((16,), jnp.float32) + jnp.sum(t)) * 0.0625
        pltpu.sync_copy(scal_v.at[pl.ds(0, 16)], out_hbm)


@jax.jit
def kernel(x, y, x_weights, y_weights):
    mesh = plsc.VectorSubcoreMesh(core_axis_name="c", subcore_axis_name="s",
                                  num_cores=1)
    run = pl.kernel(
        _body,
        out_type=jax.ShapeDtypeStruct((16,), jnp.float32),
        mesh=mesh,
        compiler_params=pltpu.CompilerParams(needs_layout_passes=False),
        scratch_types=[
            pltpu.VMEM((CHUNK + 16,), jnp.int32),
            pltpu.VMEM((CHUNK,), jnp.float32),
            pltpu.VMEM((CHUNK // 128, 128), jnp.int32),
            pltpu.VMEM((CHUNK,), jnp.int32),
            pltpu.VMEM((CHUNK,), jnp.float32),
            pltpu.VMEM((RADIX * 16,), jnp.int32),
            pltpu.VMEM((RADIX,), jnp.int32),
            pltpu.VMEM((512,), jnp.float32),
            pltpu.VMEM((512,), jnp.int32),
            pltpu.VMEM((16,), jnp.float32),
            pltpu.VMEM_SHARED((N2 + PAD,), jnp.int32),
            pltpu.VMEM_SHARED((N2,), jnp.float32),
            pltpu.VMEM_SHARED((NW * 32,), jnp.int32),
            pltpu.VMEM_SHARED((NW * 16,), jnp.float32),
            pltpu.SemaphoreType.DMA,
        ],
    )
    out = run(x, y, x_weights, y_weights)
    return out[0].reshape(())
